# baseline (speedup 1.0000x reference)
import functools

import jax
import jax.numpy as jnp
from jax.experimental import pallas as pl
from jax.experimental.pallas import tpu as pltpu

_NUM_Q = 4
_COMMIT_W = 0.25
_N_CHUNKS = 2


def _rvq_kernel(x_ref, cb_ref, q_ref, idx_ref, loss_ref,
                hi_ref, mid_ref, lo_ref, c2_ref, *, inv_count):
    i = pl.program_id(0)
    nsteps = pl.num_programs(0)

    @pl.when(i == 0)
    def _split():
        cb = cb_ref[...]
        hi = cb.astype(jnp.bfloat16)
        rem1 = cb - hi.astype(jnp.float32)
        mid = rem1.astype(jnp.bfloat16)
        rem2 = rem1 - mid.astype(jnp.float32)
        hi_ref[...] = hi
        mid_ref[...] = mid
        lo_ref[...] = rem2.astype(jnp.bfloat16)
        c2_ref[...] = jnp.sum(cb * cb, axis=2)
        loss_ref[0, 0] = jnp.float32(0.0)

    M = x_ref.shape[0]
    K = cb_ref.shape[1]
    NC = _N_CHUNKS
    H = M // NC
    lane_iota = jax.lax.broadcasted_iota(jnp.int32, (H, K), 1)
    rs = [x_ref[c * H:(c + 1) * H, :] for c in range(NC)]
    idx_cols = [[] for _ in range(NC)]
    loss_part = jnp.float32(0.0)
    for q in range(_NUM_Q):
        for h in range(NC):
            r = rs[h]
            r2 = jnp.sum(r * r, axis=1, keepdims=True)
            scores = jax.lax.dot_general(
                r.astype(jnp.bfloat16), hi_ref[q], (((1,), (1,)), ((), ())),
                preferred_element_type=jnp.float32)
            d2 = r2 - 2.0 * scores + c2_ref[q][None, :]
            minv = jnp.min(d2, axis=1, keepdims=True)
            idx = jnp.min(jnp.where(d2 == minv, lane_iota, K), axis=1)
            idx_cols[h].append(idx[:, None])
            onehot = jnp.where(lane_iota == idx[:, None],
                               jnp.float32(1),
                               jnp.float32(0)).astype(jnp.bfloat16)
            quant = jnp.float32(0.0)
            for part_ref in (hi_ref, mid_ref, lo_ref):
                quant = quant + jax.lax.dot_general(
                    onehot, part_ref[q], (((1,), (0,)), ((), ())),
                    preferred_element_type=jnp.float32)
            rs[h] = r - quant
            loss_part = loss_part + jnp.sum(rs[h] * rs[h])
    for c in range(NC):
        q_ref[c * H:(c + 1) * H, :] = x_ref[c * H:(c + 1) * H, :] - rs[c]
        idx_ref[c * H:(c + 1) * H, :] = jnp.concatenate(idx_cols[c], axis=1)

    loss_ref[0, 0] += loss_part

    @pl.when(i == nsteps - 1)
    def _finish():
        loss_ref[0, 0] = loss_ref[0, 0] * jnp.float32(_COMMIT_W * inv_count)


def _rvq_call(xs, cb, *, full_count):
    Ts, D = xs.shape
    K = cb.shape[1]
    M = 2304
    return pl.pallas_call(
        functools.partial(_rvq_kernel, inv_count=1.0 / full_count),
        grid=(Ts // M,),
        in_specs=[
            pl.BlockSpec((M, D), lambda i: (i, 0)),
            pl.BlockSpec((_NUM_Q, K, D), lambda i: (0, 0, 0)),
        ],
        out_specs=[
            pl.BlockSpec((M, D), lambda i: (i, 0)),
            pl.BlockSpec((M, _NUM_Q), lambda i: (i, 0)),
            pl.BlockSpec((1, 1), lambda i: (0, 0), memory_space=pltpu.SMEM),
        ],
        out_shape=[
            jax.ShapeDtypeStruct((Ts, D), jnp.float32),
            jax.ShapeDtypeStruct((Ts, _NUM_Q), jnp.int32),
            jax.ShapeDtypeStruct((1, 1), jnp.float32),
        ],
        scratch_shapes=[
            pltpu.VMEM((_NUM_Q, K, D), jnp.bfloat16),
            pltpu.VMEM((_NUM_Q, K, D), jnp.bfloat16),
            pltpu.VMEM((_NUM_Q, K, D), jnp.bfloat16),
            pltpu.VMEM((_NUM_Q, K), jnp.float32),
        ],
    )(xs, cb)


def _kernel_fused(x, codebooks):
    B, S, D = x.shape
    T = B * S
    qf, idxf, loss = _rvq_call(x.reshape(T, D), codebooks,
                               full_count=float(T * D))
    return qf.reshape(B, S, D), idxf.reshape(B, S, _NUM_Q), loss[0, 0]


def _argmin_body(r, hi_ref, c2_ref, lane_iota, K):
    r2 = jnp.sum(r * r, axis=1, keepdims=True)
    scores = jax.lax.dot_general(
        r.astype(jnp.bfloat16), hi_ref[...], (((1,), (1,)), ((), ())),
        preferred_element_type=jnp.float32)
    d2 = r2 - 2.0 * scores + c2_ref[0:1, :]
    minv = jnp.min(d2, axis=1, keepdims=True)
    return jnp.min(jnp.where(d2 == minv, lane_iota, K), axis=1)


def _argmin_first_kernel(x_ref, cb_ref, idx_ref, hi_ref, c2_ref):
    i = pl.program_id(0)

    @pl.when(i == 0)
    def _prep():
        cb = cb_ref[...]
        hi_ref[...] = cb.astype(jnp.bfloat16)
        c2_ref[...] = jnp.sum(cb * cb, axis=1)[None, :]

    K = cb_ref.shape[0]
    lane_iota = jax.lax.broadcasted_iota(jnp.int32, (x_ref.shape[0], K), 1)
    idx_ref[...] = _argmin_body(x_ref[...], hi_ref, c2_ref, lane_iota, K)[:, None]


def _argmin_next_kernel(rp_ref, qt_ref, cb_ref, r_ref, idx_ref, loss_ref,
                        hi_ref, c2_ref):
    i = pl.program_id(0)

    @pl.when(i == 0)
    def _prep():
        cb = cb_ref[...]
        hi_ref[...] = cb.astype(jnp.bfloat16)
        c2_ref[...] = jnp.sum(cb * cb, axis=1)[None, :]
        loss_ref[0, 0] = jnp.float32(0.0)

    K = cb_ref.shape[0]
    r = rp_ref[...] - qt_ref[...]
    r_ref[...] = r
    loss_ref[0, 0] += jnp.sum(r * r)
    lane_iota = jax.lax.broadcasted_iota(jnp.int32, (r.shape[0], K), 1)
    idx_ref[...] = _argmin_body(r, hi_ref, c2_ref, lane_iota, K)[:, None]


def _epilogue_kernel(x_ref, rp_ref, qt_ref, qout_ref, loss_ref):
    i = pl.program_id(0)

    @pl.when(i == 0)
    def _prep():
        loss_ref[0, 0] = jnp.float32(0.0)

    r = rp_ref[...] - qt_ref[...]
    loss_ref[0, 0] += jnp.sum(r * r)
    qout_ref[...] = x_ref[...] - r


_AM_M = 2304


def _argmin_first_call(xf, cb):
    T, D = xf.shape
    K = cb.shape[0]
    return pl.pallas_call(
        _argmin_first_kernel,
        grid=(T // _AM_M,),
        in_specs=[pl.BlockSpec((_AM_M, D), lambda i: (i, 0)),
                  pl.BlockSpec((K, D), lambda i: (0, 0))],
        out_specs=[pl.BlockSpec((_AM_M, 1), lambda i: (i, 0))],
        out_shape=[jax.ShapeDtypeStruct((T, 1), jnp.int32)],
        scratch_shapes=[pltpu.VMEM((K, D), jnp.bfloat16),
                        pltpu.VMEM((1, K), jnp.float32)],
    )(xf, cb)


def _argmin_next_call(rp, qt, cb):
    T, D = rp.shape
    K = cb.shape[0]
    return pl.pallas_call(
        _argmin_next_kernel,
        grid=(T // _AM_M,),
        in_specs=[pl.BlockSpec((_AM_M, D), lambda i: (i, 0)),
                  pl.BlockSpec((_AM_M, D), lambda i: (i, 0)),
                  pl.BlockSpec((K, D), lambda i: (0, 0))],
        out_specs=[pl.BlockSpec((_AM_M, D), lambda i: (i, 0)),
                   pl.BlockSpec((_AM_M, 1), lambda i: (i, 0)),
                   pl.BlockSpec((1, 1), lambda i: (0, 0),
                                memory_space=pltpu.SMEM)],
        out_shape=[jax.ShapeDtypeStruct((T, D), jnp.float32),
                   jax.ShapeDtypeStruct((T, 1), jnp.int32),
                   jax.ShapeDtypeStruct((1, 1), jnp.float32)],
        scratch_shapes=[pltpu.VMEM((K, D), jnp.bfloat16),
                        pltpu.VMEM((1, K), jnp.float32)],
    )(rp, qt, cb)


def _epilogue_call(xf, rp, qt):
    T, D = xf.shape
    return pl.pallas_call(
        _epilogue_kernel,
        grid=(T // _AM_M,),
        in_specs=[pl.BlockSpec((_AM_M, D), lambda i: (i, 0)),
                  pl.BlockSpec((_AM_M, D), lambda i: (i, 0)),
                  pl.BlockSpec((_AM_M, D), lambda i: (i, 0))],
        out_specs=[pl.BlockSpec((_AM_M, D), lambda i: (i, 0)),
                   pl.BlockSpec((1, 1), lambda i: (0, 0),
                                memory_space=pltpu.SMEM)],
        out_shape=[jax.ShapeDtypeStruct((T, D), jnp.float32),
                   jax.ShapeDtypeStruct((1, 1), jnp.float32)],
    )(xf, rp, qt)


def _sc_gather(table, idx):
    from jax.experimental.pallas import tpu_sc as plsc
    info = plsc.get_sparse_core_info()
    nw = info.num_cores * info.num_subcores
    B = idx.shape[0]
    D = table.shape[1]
    b_per_w = B // nw
    mesh = plsc.VectorSubcoreMesh(core_axis_name="c", subcore_axis_name="s")

    @functools.partial(
        pl.kernel, mesh=mesh,
        out_type=jax.ShapeDtypeStruct((B, D), jnp.float32),
        scratch_types=[
            pltpu.VMEM((b_per_w,), jnp.int32),
            pltpu.VMEM((b_per_w, D), jnp.float32),
            pltpu.SemaphoreType.DMA,
        ],
    )
    def k(table_hbm, idx_hbm, out_hbm, idx_v, rows_v, sem):
        wid = (jax.lax.axis_index("s") * info.num_cores
               + jax.lax.axis_index("c"))
        base = wid * b_per_w
        pltpu.sync_copy(idx_hbm.at[pl.ds(base, b_per_w)], idx_v)
        pltpu.async_copy(table_hbm.at[idx_v], rows_v, sem).wait()
        pltpu.sync_copy(rows_v, out_hbm.at[pl.ds(base, b_per_w)])

    return k(table, idx)


def _kernel_hybrid(x, codebooks):
    B, S, D = x.shape
    T = B * S
    xf = x.reshape(T, D)
    idx0, = _argmin_first_call(xf, codebooks[0])
    quant = _sc_gather(codebooks[0], idx0.reshape(T))
    r_prev = xf
    idxs = [idx0]
    loss_parts = []
    for q in range(1, _NUM_Q):
        r_prev, idxq, lq = _argmin_next_call(r_prev, quant, codebooks[q])
        quant = _sc_gather(codebooks[q], idxq.reshape(T))
        idxs.append(idxq)
        loss_parts.append(lq[0, 0])
    qout, l_last = _epilogue_call(xf, r_prev, quant)
    loss = (sum(loss_parts) + l_last[0, 0]) * jnp.float32(
        _COMMIT_W / float(T * D))
    indices = jnp.concatenate(idxs, axis=1).reshape(B, S, _NUM_Q)
    return qout.reshape(B, S, D), indices, loss


def kernel(x, codebooks):
    return _kernel_hybrid(x, codebooks)


# speedup vs baseline: 1.9207x; 1.9207x over previous
import functools

import jax
import jax.numpy as jnp
from jax.experimental import pallas as pl
from jax.experimental.pallas import tpu as pltpu

_NUM_Q = 4
_COMMIT_W = 0.25
_N_CHUNKS = 2


def _rvq_kernel(x_ref, cb_ref, q_ref, idx_ref, loss_ref,
                hi_ref, mid_ref, lo_ref, c2_ref, *, inv_count):
    i = pl.program_id(0)
    nsteps = pl.num_programs(0)

    @pl.when(i == 0)
    def _split():
        cb = cb_ref[...]
        hi = cb.astype(jnp.bfloat16)
        rem1 = cb - hi.astype(jnp.float32)
        mid = rem1.astype(jnp.bfloat16)
        rem2 = rem1 - mid.astype(jnp.float32)
        hi_ref[...] = hi
        mid_ref[...] = mid
        lo_ref[...] = rem2.astype(jnp.bfloat16)
        c2_ref[...] = jnp.sum(cb * cb, axis=2)
        loss_ref[0, 0] = jnp.float32(0.0)

    M = x_ref.shape[0]
    K = cb_ref.shape[1]
    NC = _N_CHUNKS
    H = M // NC
    lane_iota = jax.lax.broadcasted_iota(
        jnp.int32, (H, K), 1).astype(jnp.float32)
    rs = [x_ref[c * H:(c + 1) * H, :] for c in range(NC)]
    idx_cols = [[] for _ in range(NC)]
    loss_part = jnp.float32(0.0)
    for q in range(_NUM_Q):
        for h in range(NC):
            r = rs[h]
            r2 = jnp.sum(r * r, axis=1, keepdims=True)
            scores = jax.lax.dot_general(
                r.astype(jnp.bfloat16), hi_ref[q], (((1,), (1,)), ((), ())),
                preferred_element_type=jnp.float32)
            d2 = r2 - 2.0 * scores + c2_ref[q][None, :]
            minv = jnp.min(d2, axis=1, keepdims=True)
            idx = jnp.min(jnp.where(d2 == minv, lane_iota,
                                    jnp.float32(K)), axis=1)
            idx_cols[h].append(idx.astype(jnp.int32)[:, None])
            onehot = jnp.where(lane_iota == idx[:, None],
                               jnp.float32(1),
                               jnp.float32(0)).astype(jnp.bfloat16)
            quant = jnp.float32(0.0)
            for part_ref in (hi_ref, mid_ref, lo_ref):
                quant = quant + jax.lax.dot_general(
                    onehot, part_ref[q], (((1,), (0,)), ((), ())),
                    preferred_element_type=jnp.float32)
            rs[h] = r - quant
            loss_part = loss_part + jnp.sum(rs[h] * rs[h])
    for c in range(NC):
        q_ref[c * H:(c + 1) * H, :] = x_ref[c * H:(c + 1) * H, :] - rs[c]
        idx_ref[c * H:(c + 1) * H, :] = jnp.concatenate(idx_cols[c], axis=1)

    loss_ref[0, 0] += loss_part

    @pl.when(i == nsteps - 1)
    def _finish():
        loss_ref[0, 0] = loss_ref[0, 0] * jnp.float32(_COMMIT_W * inv_count)


def _rvq_call(xs, cb, *, full_count):
    Ts, D = xs.shape
    K = cb.shape[1]
    M = 2304
    return pl.pallas_call(
        functools.partial(_rvq_kernel, inv_count=1.0 / full_count),
        grid=(Ts // M,),
        in_specs=[
            pl.BlockSpec((M, D), lambda i: (i, 0)),
            pl.BlockSpec((_NUM_Q, K, D), lambda i: (0, 0, 0)),
        ],
        out_specs=[
            pl.BlockSpec((M, D), lambda i: (i, 0)),
            pl.BlockSpec((M, _NUM_Q), lambda i: (i, 0)),
            pl.BlockSpec((1, 1), lambda i: (0, 0), memory_space=pltpu.SMEM),
        ],
        out_shape=[
            jax.ShapeDtypeStruct((Ts, D), jnp.float32),
            jax.ShapeDtypeStruct((Ts, _NUM_Q), jnp.int32),
            jax.ShapeDtypeStruct((1, 1), jnp.float32),
        ],
        scratch_shapes=[
            pltpu.VMEM((_NUM_Q, K, D), jnp.bfloat16),
            pltpu.VMEM((_NUM_Q, K, D), jnp.bfloat16),
            pltpu.VMEM((_NUM_Q, K, D), jnp.bfloat16),
            pltpu.VMEM((_NUM_Q, K), jnp.float32),
        ],
    )(xs, cb)


def _kernel_fused(x, codebooks):
    B, S, D = x.shape
    T = B * S
    qf, idxf, loss = _rvq_call(x.reshape(T, D), codebooks,
                               full_count=float(T * D))
    return qf.reshape(B, S, D), idxf.reshape(B, S, _NUM_Q), loss[0, 0]


def _argmin_body(r, hi_ref, c2_ref, lane_iota, K):
    r2 = jnp.sum(r * r, axis=1, keepdims=True)
    scores = jax.lax.dot_general(
        r.astype(jnp.bfloat16), hi_ref[...], (((1,), (1,)), ((), ())),
        preferred_element_type=jnp.float32)
    d2 = r2 - 2.0 * scores + c2_ref[0:1, :]
    minv = jnp.min(d2, axis=1, keepdims=True)
    return jnp.min(jnp.where(d2 == minv, lane_iota, K), axis=1)


def _argmin_first_kernel(x_ref, cb_ref, idx_ref, hi_ref, c2_ref):
    i = pl.program_id(0)

    @pl.when(i == 0)
    def _prep():
        cb = cb_ref[...]
        hi_ref[...] = cb.astype(jnp.bfloat16)
        c2_ref[...] = jnp.sum(cb * cb, axis=1)[None, :]

    K = cb_ref.shape[0]
    lane_iota = jax.lax.broadcasted_iota(jnp.int32, (x_ref.shape[0], K), 1)
    idx_ref[...] = _argmin_body(x_ref[...], hi_ref, c2_ref, lane_iota, K)[:, None]


def _argmin_next_kernel(rp_ref, qt_ref, cb_ref, r_ref, idx_ref, loss_ref,
                        hi_ref, c2_ref):
    i = pl.program_id(0)

    @pl.when(i == 0)
    def _prep():
        cb = cb_ref[...]
        hi_ref[...] = cb.astype(jnp.bfloat16)
        c2_ref[...] = jnp.sum(cb * cb, axis=1)[None, :]
        loss_ref[0, 0] = jnp.float32(0.0)

    K = cb_ref.shape[0]
    r = rp_ref[...] - qt_ref[...]
    r_ref[...] = r
    loss_ref[0, 0] += jnp.sum(r * r)
    lane_iota = jax.lax.broadcasted_iota(jnp.int32, (r.shape[0], K), 1)
    idx_ref[...] = _argmin_body(r, hi_ref, c2_ref, lane_iota, K)[:, None]


def _epilogue_kernel(x_ref, rp_ref, qt_ref, qout_ref, loss_ref):
    i = pl.program_id(0)

    @pl.when(i == 0)
    def _prep():
        loss_ref[0, 0] = jnp.float32(0.0)

    r = rp_ref[...] - qt_ref[...]
    loss_ref[0, 0] += jnp.sum(r * r)
    qout_ref[...] = x_ref[...] - r


_AM_M = 2304


def _argmin_first_call(xf, cb):
    T, D = xf.shape
    K = cb.shape[0]
    return pl.pallas_call(
        _argmin_first_kernel,
        grid=(T // _AM_M,),
        in_specs=[pl.BlockSpec((_AM_M, D), lambda i: (i, 0)),
                  pl.BlockSpec((K, D), lambda i: (0, 0))],
        out_specs=[pl.BlockSpec((_AM_M, 1), lambda i: (i, 0))],
        out_shape=[jax.ShapeDtypeStruct((T, 1), jnp.int32)],
        scratch_shapes=[pltpu.VMEM((K, D), jnp.bfloat16),
                        pltpu.VMEM((1, K), jnp.float32)],
    )(xf, cb)


def _argmin_next_call(rp, qt, cb):
    T, D = rp.shape
    K = cb.shape[0]
    return pl.pallas_call(
        _argmin_next_kernel,
        grid=(T // _AM_M,),
        in_specs=[pl.BlockSpec((_AM_M, D), lambda i: (i, 0)),
                  pl.BlockSpec((_AM_M, D), lambda i: (i, 0)),
                  pl.BlockSpec((K, D), lambda i: (0, 0))],
        out_specs=[pl.BlockSpec((_AM_M, D), lambda i: (i, 0)),
                   pl.BlockSpec((_AM_M, 1), lambda i: (i, 0)),
                   pl.BlockSpec((1, 1), lambda i: (0, 0),
                                memory_space=pltpu.SMEM)],
        out_shape=[jax.ShapeDtypeStruct((T, D), jnp.float32),
                   jax.ShapeDtypeStruct((T, 1), jnp.int32),
                   jax.ShapeDtypeStruct((1, 1), jnp.float32)],
        scratch_shapes=[pltpu.VMEM((K, D), jnp.bfloat16),
                        pltpu.VMEM((1, K), jnp.float32)],
    )(rp, qt, cb)


def _epilogue_call(xf, rp, qt):
    T, D = xf.shape
    return pl.pallas_call(
        _epilogue_kernel,
        grid=(T // _AM_M,),
        in_specs=[pl.BlockSpec((_AM_M, D), lambda i: (i, 0)),
                  pl.BlockSpec((_AM_M, D), lambda i: (i, 0)),
                  pl.BlockSpec((_AM_M, D), lambda i: (i, 0))],
        out_specs=[pl.BlockSpec((_AM_M, D), lambda i: (i, 0)),
                   pl.BlockSpec((1, 1), lambda i: (0, 0),
                                memory_space=pltpu.SMEM)],
        out_shape=[jax.ShapeDtypeStruct((T, D), jnp.float32),
                   jax.ShapeDtypeStruct((1, 1), jnp.float32)],
    )(xf, rp, qt)


def _sc_gather(table, idx):
    from jax.experimental.pallas import tpu_sc as plsc
    info = plsc.get_sparse_core_info()
    nw = info.num_cores * info.num_subcores
    B = idx.shape[0]
    D = table.shape[1]
    b_per_w = B // nw
    mesh = plsc.VectorSubcoreMesh(core_axis_name="c", subcore_axis_name="s")

    @functools.partial(
        pl.kernel, mesh=mesh,
        out_type=jax.ShapeDtypeStruct((B, D), jnp.float32),
        scratch_types=[
            pltpu.VMEM((b_per_w,), jnp.int32),
            pltpu.VMEM((b_per_w, D), jnp.float32),
            pltpu.SemaphoreType.DMA,
        ],
    )
    def k(table_hbm, idx_hbm, out_hbm, idx_v, rows_v, sem):
        wid = (jax.lax.axis_index("s") * info.num_cores
               + jax.lax.axis_index("c"))
        base = wid * b_per_w
        pltpu.sync_copy(idx_hbm.at[pl.ds(base, b_per_w)], idx_v)
        pltpu.async_copy(table_hbm.at[idx_v], rows_v, sem).wait()
        pltpu.sync_copy(rows_v, out_hbm.at[pl.ds(base, b_per_w)])

    return k(table, idx)


def _kernel_hybrid(x, codebooks):
    B, S, D = x.shape
    T = B * S
    xf = x.reshape(T, D)
    idx0, = _argmin_first_call(xf, codebooks[0])
    quant = _sc_gather(codebooks[0], idx0.reshape(T))
    r_prev = xf
    idxs = [idx0]
    loss_parts = []
    for q in range(1, _NUM_Q):
        r_prev, idxq, lq = _argmin_next_call(r_prev, quant, codebooks[q])
        quant = _sc_gather(codebooks[q], idxq.reshape(T))
        idxs.append(idxq)
        loss_parts.append(lq[0, 0])
    qout, l_last = _epilogue_call(xf, r_prev, quant)
    loss = (sum(loss_parts) + l_last[0, 0]) * jnp.float32(
        _COMMIT_W / float(T * D))
    indices = jnp.concatenate(idxs, axis=1).reshape(B, S, _NUM_Q)
    return qout.reshape(B, S, D), indices, loss


def kernel(x, codebooks):
    return _kernel_fused(x, codebooks)
